# Initial kernel scaffold; baseline (speedup 1.0000x reference)
import functools

import jax
import jax.numpy as jnp
from jax import lax
from jax.experimental import pallas as pl
from jax.experimental.pallas import tpu as pltpu
from jax.experimental.pallas import tpu_sc as plsc

N_MODELS = 100000
D = 64
B = 16384

try:
    _info = plsc.get_sparse_core_info()
    NC, NS, L = _info.num_cores, _info.num_subcores, _info.num_lanes
except Exception:
    NC, NS, L = 2, 16, 16

NW = NC * NS
B_PER_W = B // NW
CHUNK = 128
N_CHUNKS = B_PER_W // CHUNK
N_GROUPS = B_PER_W // L

_MESH = plsc.VectorSubcoreMesh(core_axis_name="c", subcore_axis_name="s")


def _sc_kernel(idx_hbm, w_hbm, table_hbm, out_hbm,
               idx_v, rows_v, p_v, w_v, out_v, sem):
    wid = lax.axis_index("s") * NC + lax.axis_index("c")
    base = wid * B_PER_W

    pltpu.sync_copy(w_hbm, w_v)
    for j in range(N_CHUNKS):
        pltpu.sync_copy(idx_hbm.at[pl.ds(base + j * CHUNK, CHUNK)],
                        idx_v.at[j])

    copies = []
    for j in range(N_CHUNKS):
        copies.append(pltpu.async_copy(table_hbm.at[idx_v.at[j]],
                                       rows_v.at[pl.ds(j * CHUNK, CHUNK)],
                                       sem))
    for c in copies:
        c.wait()

    w0 = w_v[pl.ds(0, L)]
    w1 = w_v[pl.ds(L, L)]
    w2 = w_v[pl.ds(2 * L, L)]
    w3 = w_v[pl.ds(3 * L, L)]

    def body1(q, carry):
        acc = rows_v[q, pl.ds(0, L)] * w0
        acc = acc + rows_v[q, pl.ds(L, L)] * w1
        acc = acc + rows_v[q, pl.ds(2 * L, L)] * w2
        acc = acc + rows_v[q, pl.ds(3 * L, L)] * w3
        p_v[q, :] = acc
        return carry

    lax.fori_loop(0, B_PER_W, body1, 0, unroll=4)

    iota = lax.iota(jnp.int32, L)

    def body2(g, carry):
        row_idx = g * L + iota
        acc = plsc.load_gather(p_v, [row_idx, jnp.full((L,), 0, jnp.int32)])
        for d in range(1, L):
            acc = acc + plsc.load_gather(
                p_v, [row_idx, jnp.full((L,), d, jnp.int32)])
        out_v[pl.ds(pl.multiple_of(g * L, L), L)] = acc
        return carry

    lax.fori_loop(0, N_GROUPS, body2, 0)

    pltpu.sync_copy(out_v, out_hbm.at[pl.ds(base, B_PER_W)])


@jax.jit
def kernel(prompt_embedding, model_id, w_u, v_m_weight):
    del prompt_embedding
    run = functools.partial(
        pl.kernel,
        mesh=_MESH,
        out_type=jax.ShapeDtypeStruct((B,), jnp.float32),
        scratch_types=[
            pltpu.VMEM((N_CHUNKS, CHUNK), jnp.int32),
            pltpu.VMEM((B_PER_W, D), jnp.float32),
            pltpu.VMEM((B_PER_W, L), jnp.float32),
            pltpu.VMEM((D,), jnp.float32),
            pltpu.VMEM((B_PER_W,), jnp.float32),
            pltpu.SemaphoreType.DMA,
        ],
    )(_sc_kernel)
    return run(model_id, w_u, v_m_weight)


# speedup vs baseline: 1.0171x; 1.0171x over previous
import functools

import jax
import jax.numpy as jnp
from jax import lax
from jax.experimental import pallas as pl
from jax.experimental.pallas import tpu as pltpu
from jax.experimental.pallas import tpu_sc as plsc

N_MODELS = 100000
D = 64
B = 16384

try:
    _info = plsc.get_sparse_core_info()
    NC, NS, L = _info.num_cores, _info.num_subcores, _info.num_lanes
except Exception:
    NC, NS, L = 2, 16, 16

NW = NC * NS
B_PER_W = B // NW
CHUNK = 128
N_CHUNKS = B_PER_W // CHUNK
N_GROUPS = B_PER_W // L

_MESH = plsc.VectorSubcoreMesh(core_axis_name="c", subcore_axis_name="s")


def _sc_kernel(idx_hbm, w_hbm, table_hbm, out_hbm,
               idx_v, rows_v, w_v, out_v, sem):
    wid = lax.axis_index("s") * NC + lax.axis_index("c")
    base = wid * B_PER_W

    pltpu.sync_copy(w_hbm, w_v)
    for j in range(N_CHUNKS):
        pltpu.sync_copy(idx_hbm.at[pl.ds(base + j * CHUNK, CHUNK)],
                        idx_v.at[j])

    copies = []
    for j in range(N_CHUNKS):
        copies.append(pltpu.async_copy(table_hbm.at[idx_v.at[j]],
                                       rows_v.at[pl.ds(j * CHUNK, CHUNK)],
                                       sem))
    for c in copies:
        c.wait()

    w0 = w_v[pl.ds(0, L)]
    w1 = w_v[pl.ds(L, L)]
    w2 = w_v[pl.ds(2 * L, L)]
    w3 = w_v[pl.ds(3 * L, L)]

    iota = lax.iota(jnp.int32, L)

    def body1(g, carry):
        vec = jnp.zeros((L,), jnp.float32)
        for i in range(L):
            q = g * L + i
            acc = rows_v[q, pl.ds(0, L)] * w0
            acc = acc + rows_v[q, pl.ds(L, L)] * w1
            acc = acc + rows_v[q, pl.ds(2 * L, L)] * w2
            acc = acc + rows_v[q, pl.ds(3 * L, L)] * w3
            vec = jnp.where(iota == i, jnp.sum(acc), vec)
        out_v[pl.ds(pl.multiple_of(g * L, L), L)] = vec
        return carry

    lax.fori_loop(0, N_GROUPS, body1, 0)

    pltpu.sync_copy(out_v, out_hbm.at[pl.ds(base, B_PER_W)])


@jax.jit
def kernel(prompt_embedding, model_id, w_u, v_m_weight):
    del prompt_embedding
    run = functools.partial(
        pl.kernel,
        mesh=_MESH,
        compiler_params=pltpu.CompilerParams(
            needs_layout_passes=False, use_tc_tiling_on_sc=False),
        out_type=jax.ShapeDtypeStruct((B,), jnp.float32),
        scratch_types=[
            pltpu.VMEM((N_CHUNKS, CHUNK), jnp.int32),
            pltpu.VMEM((B_PER_W, D), jnp.float32),
            pltpu.VMEM((D,), jnp.float32),
            pltpu.VMEM((B_PER_W,), jnp.float32),
            pltpu.SemaphoreType.DMA,
        ],
    )(_sc_kernel)
    return run(model_id, w_u, v_m_weight)


# speedup vs baseline: 2.7406x; 2.6944x over previous
import functools

import jax
import jax.numpy as jnp
from jax import lax
from jax.experimental import pallas as pl
from jax.experimental.pallas import tpu as pltpu
from jax.experimental.pallas import tpu_sc as plsc

N_MODELS = 100000
D = 64
B = 16384
BLK = 16384

try:
    _info = plsc.get_sparse_core_info()
    NC, NS, L = _info.num_cores, _info.num_subcores, _info.num_lanes
except Exception:
    NC, NS, L = 2, 16, 16

NW = NC * NS
B_PER_W = B // NW
CHUNK = 128
N_CHUNKS = B_PER_W // CHUNK

_MESH = plsc.VectorSubcoreMesh(core_axis_name="c", subcore_axis_name="s")


def _tc_matvec(w_ref, at_ref, out_ref):
    out_ref[...] = jnp.sum(at_ref[...] * w_ref[...], axis=0)


def _sc_gather(idx_hbm, scores_hbm, out_hbm, idx_v, g_v, sem):
    wid = lax.axis_index("s") * NC + lax.axis_index("c")
    base = wid * B_PER_W

    for j in range(N_CHUNKS):
        pltpu.sync_copy(idx_hbm.at[pl.ds(base + j * CHUNK, CHUNK)],
                        idx_v.at[j])
    copies = [
        pltpu.async_copy(scores_hbm.at[idx_v.at[j]], g_v.at[j], sem)
        for j in range(N_CHUNKS)
    ]
    for c in copies:
        c.wait()
    for j in range(N_CHUNKS):
        pltpu.sync_copy(g_v.at[j], out_hbm.at[pl.ds(base + j * CHUNK, CHUNK)])


@jax.jit
def kernel(prompt_embedding, model_id, w_u, v_m_weight):
    del prompt_embedding

    a_t = jnp.transpose(v_m_weight)
    w2 = w_u[:, None]
    grid = pl.cdiv(N_MODELS, BLK)
    all_scores = pl.pallas_call(
        _tc_matvec,
        grid=(grid,),
        in_specs=[
            pl.BlockSpec((D, 1), lambda i: (0, 0)),
            pl.BlockSpec((D, BLK), lambda i: (0, i)),
        ],
        out_specs=pl.BlockSpec((BLK,), lambda i: (i,)),
        out_shape=jax.ShapeDtypeStruct((N_MODELS,), jnp.float32),
    )(w2, a_t)

    run = functools.partial(
        pl.kernel,
        mesh=_MESH,
        compiler_params=pltpu.CompilerParams(
            needs_layout_passes=False, use_tc_tiling_on_sc=False),
        out_type=jax.ShapeDtypeStruct((B,), jnp.float32),
        scratch_types=[
            pltpu.VMEM((N_CHUNKS, CHUNK), jnp.int32),
            pltpu.VMEM((N_CHUNKS, CHUNK), jnp.float32),
            pltpu.SemaphoreType.DMA,
        ],
    )(_sc_gather)
    return run(model_id, all_scores)
